# baseline (speedup 1.0000x reference)
import functools

import jax
import jax.numpy as jnp
from jax import lax
from jax.experimental import pallas as pl
from jax.experimental.pallas import tpu as pltpu
from jax.experimental.pallas import tpu_sc as plsc

_NC = 2
_NS = 16
_L = 16
_NW = _NC * _NS

_H = 224
_W = 224
_NIMG = 768
_NSC = 64
_KPER = _NSC // _NW
_IB = 32


def _sc_partials(gt_hbm, pr_hbm, sum_out, cnt_out, gbuf, pbuf, vrow,
                 sacc_v, cacc_v):
    wid = lax.axis_index("s") * _NC + lax.axis_index("c")

    nb = _W // _L

    def image_body(j, carry):
        sacc, cacc = carry
        img = (_NIMG - _NSC) + wid * _KPER + j
        pltpu.sync_copy(gt_hbm.at[img], gbuf)
        pltpu.sync_copy(pr_hbm.at[img], pbuf)

        def row_body(r, rcarry):
            sa, ca = rcarry
            for b in range(nb):
                c0 = b * _L
                g0 = gbuf[r - 1, pl.ds(c0, _L)]
                g1 = gbuf[r, pl.ds(c0, _L)]
                g2 = gbuf[r + 1, pl.ds(c0, _L)]
                vrow[pl.ds(c0 + 1, _L)] = jnp.maximum(g0, jnp.maximum(g1, g2))
            for b in range(nb):
                c0 = b * _L
                vl = vrow[pl.ds(c0, _L)]
                vc = vrow[pl.ds(c0 + 1, _L)]
                vr = vrow[pl.ds(c0 + 2, _L)]
                wm = jnp.maximum(vl, jnp.maximum(vc, vr))
                gc = gbuf[r, pl.ds(c0, _L)]
                pc = pbuf[r, pl.ds(c0, _L)]
                m = (wm <= gc) & (gc > 0.0)
                if b == 0:
                    m = m & (lax.iota(jnp.int32, _L) >= 1)
                elif b == nb - 1:
                    m = m & (lax.iota(jnp.int32, _L) <= _L - 2)
                sa = sa + jnp.where(m, jnp.abs(pc - gc), 0.0)
                ca = ca + jnp.where(m, 1.0, 0.0)
            return sa, ca

        return lax.fori_loop(1, _H - 1, row_body, (sacc, cacc))

    zero = jnp.zeros((_L,), jnp.float32)
    sacc, cacc = lax.fori_loop(0, _KPER, image_body, (zero, zero))
    sacc_v[...] = sacc
    cacc_v[...] = cacc
    pltpu.sync_copy(sacc_v, sum_out.at[wid])
    pltpu.sync_copy(cacc_v, cnt_out.at[wid])


def _tc_partials(gt_ref, pr_ref, out_ref, s_ref, c_ref):
    i = pl.program_id(0)

    @pl.when(i == 0)
    def _init():
        s_ref[0] = 0.0
        c_ref[0] = 0.0

    g = gt_ref[...]
    p = pr_ref[...]
    ib, h, w = g.shape
    m2v = jnp.maximum(g[:, :-1, :], g[:, 1:, :])
    v3 = jnp.maximum(m2v[:, :-1, :], m2v[:, 1:, :])
    vp = jnp.pad(v3, ((0, 0), (1, 1), (0, 256 - w)),
                 constant_values=jnp.inf)
    wm = jnp.maximum(pltpu.roll(vp, 1, 2),
                     jnp.maximum(vp, pltpu.roll(vp, 255, 2)))[:, :, :w]
    mask = (wm == g) & (g > 0.0)
    s_ref[0] += jnp.sum(jnp.where(mask, jnp.abs(p - g), 0.0))
    c_ref[0] += jnp.sum(jnp.where(mask, 1.0, 0.0))

    @pl.when(i == pl.num_programs(0) - 1)
    def _fin():
        out_ref[0] = s_ref[0]
        out_ref[1] = c_ref[0]


def _combine(sc_s_ref, sc_c_ref, tc_ref, out_ref):
    s = jnp.sum(sc_s_ref[...]) + tc_ref[0]
    c = jnp.sum(sc_c_ref[...]) + tc_ref[1]
    out_ref[0] = s / (c + 0.0001)


def kernel(predict, gt):
    g3 = gt.reshape(_NIMG, _H, _W)
    p3 = predict.reshape(_NIMG, _H, _W)

    sc = functools.partial(
        pl.kernel,
        mesh=plsc.VectorSubcoreMesh(core_axis_name="c", subcore_axis_name="s"),
        out_type=[
            jax.ShapeDtypeStruct((_NW, _L), jnp.float32),
            jax.ShapeDtypeStruct((_NW, _L), jnp.float32),
        ],
        scratch_types=[
            pltpu.VMEM((_H, _W), jnp.float32),
            pltpu.VMEM((_H, _W), jnp.float32),
            pltpu.VMEM((_W + 2,), jnp.float32),
            pltpu.VMEM((_L,), jnp.float32),
            pltpu.VMEM((_L,), jnp.float32),
        ],
    )(_sc_partials)
    sc_sums, sc_cnts = sc(g3, p3)

    ntc = _NIMG - _NSC
    tc_part = pl.pallas_call(
        _tc_partials,
        grid=(ntc // _IB,),
        in_specs=[
            pl.BlockSpec((_IB, _H, _W), lambda i: (i, 0, 0)),
            pl.BlockSpec((_IB, _H, _W), lambda i: (i, 0, 0)),
        ],
        out_specs=pl.BlockSpec(memory_space=pltpu.SMEM),
        out_shape=jax.ShapeDtypeStruct((2,), jnp.float32),
        scratch_shapes=[
            pltpu.SMEM((1,), jnp.float32),
            pltpu.SMEM((1,), jnp.float32),
        ],
    )(g3, p3)

    loss = pl.pallas_call(
        _combine,
        in_specs=[
            pl.BlockSpec((_NW, _L), lambda: (0, 0)),
            pl.BlockSpec((_NW, _L), lambda: (0, 0)),
            pl.BlockSpec(memory_space=pltpu.SMEM),
        ],
        out_specs=pl.BlockSpec(memory_space=pltpu.SMEM),
        out_shape=jax.ShapeDtypeStruct((1,), jnp.float32),
    )(sc_sums, sc_cnts, tc_part)
    return loss[0]


# speedup vs baseline: 1.0220x; 1.0220x over previous
import functools

import jax
import jax.numpy as jnp
from jax.experimental import pallas as pl
from jax.experimental.pallas import tpu as pltpu

_IB = 16


def _loss_block(gt_ref, pr_ref, out_ref, s_ref, c_ref):
    i = pl.program_id(0)

    @pl.when(i == 0)
    def _init():
        s_ref[0] = 0.0
        c_ref[0] = 0.0

    g = gt_ref[...]
    p = pr_ref[...]
    ib, h, w = g.shape
    m2v = jnp.maximum(g[:, :-1, :], g[:, 1:, :])
    v3 = jnp.maximum(m2v[:, :-1, :], m2v[:, 1:, :])
    vp = jnp.pad(v3, ((0, 0), (1, 1), (0, 256 - w)),
                 constant_values=jnp.inf)
    wm = jnp.maximum(pltpu.roll(vp, 1, 2),
                     jnp.maximum(vp, pltpu.roll(vp, 255, 2)))[:, :, :w]
    mask = (wm == g) & (g > 0.0)
    s_ref[0] += jnp.sum(jnp.where(mask, jnp.abs(p - g), 0.0))
    c_ref[0] += jnp.sum(jnp.where(mask, 1.0, 0.0))

    @pl.when(i == pl.num_programs(0) - 1)
    def _fin():
        out_ref[0] = s_ref[0] / (c_ref[0] + 0.0001)


def kernel(predict, gt):
    n = gt.shape[0] * gt.shape[1]
    h, w = gt.shape[2], gt.shape[3]
    g3 = gt.reshape(n, h, w)
    p3 = predict.reshape(n, h, w)
    grid = (n // _IB,)
    loss = pl.pallas_call(
        _loss_block,
        grid=grid,
        in_specs=[
            pl.BlockSpec((_IB, h, w), lambda i: (i, 0, 0)),
            pl.BlockSpec((_IB, h, w), lambda i: (i, 0, 0)),
        ],
        out_specs=pl.BlockSpec(memory_space=pltpu.SMEM),
        out_shape=jax.ShapeDtypeStruct((1,), jnp.float32),
        scratch_shapes=[
            pltpu.SMEM((1,), jnp.float32),
            pltpu.SMEM((1,), jnp.float32),
        ],
    )(g3, p3)
    return loss[0]


# speedup vs baseline: 1.0993x; 1.0756x over previous
import functools

import jax
import jax.numpy as jnp
from jax.experimental import pallas as pl
from jax.experimental.pallas import tpu as pltpu

_IB = 48


def _loss_block(gt_ref, pr_ref, out_ref, s_ref, c_ref):
    i = pl.program_id(0)

    @pl.when(i == 0)
    def _init():
        s_ref[0] = 0.0
        c_ref[0] = 0.0

    g = gt_ref[...]
    p = pr_ref[...]
    ib, h, w = g.shape
    v3 = jnp.maximum(g[:, :-2, :], jnp.maximum(g[:, 1:-1, :], g[:, 2:, :]))
    vp = jnp.pad(v3, ((0, 0), (1, 1), (0, 256 - w)),
                 constant_values=jnp.inf)
    wm = jnp.maximum(pltpu.roll(vp, 1, 2),
                     jnp.maximum(vp, pltpu.roll(vp, 255, 2)))[:, :, :w]
    mask = (wm == g) & (g > 0.0)
    s_ref[0] += jnp.sum(jnp.where(mask, jnp.abs(p - g), 0.0))
    c_ref[0] += jnp.sum(jnp.where(mask, 1.0, 0.0))

    @pl.when(i == pl.num_programs(0) - 1)
    def _fin():
        out_ref[0] = s_ref[0] / (c_ref[0] + 0.0001)


def kernel(predict, gt):
    n = gt.shape[0] * gt.shape[1]
    h, w = gt.shape[2], gt.shape[3]
    g3 = gt.reshape(n, h, w)
    p3 = predict.reshape(n, h, w)
    grid = (n // _IB,)
    loss = pl.pallas_call(
        _loss_block,
        grid=grid,
        in_specs=[
            pl.BlockSpec((_IB, h, w), lambda i: (i, 0, 0)),
            pl.BlockSpec((_IB, h, w), lambda i: (i, 0, 0)),
        ],
        out_specs=pl.BlockSpec(memory_space=pltpu.SMEM),
        out_shape=jax.ShapeDtypeStruct((1,), jnp.float32),
        compiler_params=pltpu.CompilerParams(
            vmem_limit_bytes=120 * 1024 * 1024),
        scratch_shapes=[
            pltpu.SMEM((1,), jnp.float32),
            pltpu.SMEM((1,), jnp.float32),
        ],
    )(g3, p3)
    return loss[0]


# speedup vs baseline: 1.0995x; 1.0002x over previous
import functools

import jax
import jax.numpy as jnp
from jax.experimental import pallas as pl
from jax.experimental.pallas import tpu as pltpu

_IB = 32


def _loss_block(gt_ref, pr_ref, out_ref, s_ref, c_ref):
    i = pl.program_id(0)

    @pl.when(i == 0)
    def _init():
        s_ref[0] = 0.0
        c_ref[0] = 0.0

    g = gt_ref[...]
    p = pr_ref[...]
    ib, h, w = g.shape
    m2v = jnp.maximum(g[:, :-1, :], g[:, 1:, :])
    v3 = jnp.maximum(m2v[:, :-1, :], m2v[:, 1:, :])
    vp = jnp.pad(v3, ((0, 0), (1, 1), (0, 256 - w)),
                 constant_values=jnp.inf)
    wm = jnp.maximum(pltpu.roll(vp, 1, 2),
                     jnp.maximum(vp, pltpu.roll(vp, 255, 2)))[:, :, :w]
    mask = (wm == g) & (g > 0.0)
    s_ref[0] += jnp.sum(jnp.where(mask, jnp.abs(p - g), 0.0))
    c_ref[0] += jnp.sum(jnp.where(mask, 1.0, 0.0))

    @pl.when(i == pl.num_programs(0) - 1)
    def _fin():
        out_ref[0] = s_ref[0] / (c_ref[0] + 0.0001)


def kernel(predict, gt):
    n = gt.shape[0] * gt.shape[1]
    h, w = gt.shape[2], gt.shape[3]
    g3 = gt.reshape(n, h, w)
    p3 = predict.reshape(n, h, w)
    grid = (n // _IB,)
    loss = pl.pallas_call(
        _loss_block,
        grid=grid,
        in_specs=[
            pl.BlockSpec((_IB, h, w), lambda i: (i, 0, 0)),
            pl.BlockSpec((_IB, h, w), lambda i: (i, 0, 0)),
        ],
        out_specs=pl.BlockSpec(memory_space=pltpu.SMEM),
        out_shape=jax.ShapeDtypeStruct((1,), jnp.float32),
        scratch_shapes=[
            pltpu.SMEM((1,), jnp.float32),
            pltpu.SMEM((1,), jnp.float32),
        ],
    )(g3, p3)
    return loss[0]
